# baseline (speedup 1.0000x reference)
import functools

import jax
import jax.numpy as jnp
from jax import lax
from jax.experimental import pallas as pl
from jax.experimental.pallas import tpu as pltpu
from jax.experimental.pallas import tpu_sc as plsc

B = 16384
D_GMF = 32
D_MLP = 64
NG = 26
GP = 16
H1 = 128
H2 = 64

_NC = 2
_NS = 16
_NW = _NC * _NS
_BPW = B // _NW


def _sc_gather(user_ids, item_ids, user_gmf, item_gmf, user_mlp, item_mlp):
    mesh = plsc.VectorSubcoreMesh(core_axis_name="c", subcore_axis_name="s")

    @functools.partial(
        pl.kernel,
        mesh=mesh,
        compiler_params=pltpu.CompilerParams(use_tc_tiling_on_sc=False),
        out_type=[
            jax.ShapeDtypeStruct((B, D_GMF), jnp.float32),
            jax.ShapeDtypeStruct((B, D_GMF), jnp.float32),
            jax.ShapeDtypeStruct((B, D_MLP), jnp.float32),
            jax.ShapeDtypeStruct((B, D_MLP), jnp.float32),
        ],
        scratch_types=[
            pltpu.VMEM((_BPW,), jnp.int32),
            pltpu.VMEM((_BPW,), jnp.int32),
            pltpu.VMEM((_BPW, D_GMF), jnp.float32),
            pltpu.VMEM((_BPW, D_GMF), jnp.float32),
            pltpu.VMEM((_BPW, D_MLP), jnp.float32),
            pltpu.VMEM((_BPW, D_MLP), jnp.float32),
            pltpu.SemaphoreType.DMA,
        ],
    )
    def gather_kernel(uid_hbm, iid_hbm, ug_hbm, ig_hbm, um_hbm, im_hbm,
                      out_ug, out_ig, out_um, out_im,
                      uidx_v, iidx_v, ug_v, ig_v, um_v, im_v, sem):
        wid = lax.axis_index("s") * _NC + lax.axis_index("c")
        base = wid * _BPW
        pltpu.sync_copy(uid_hbm.at[pl.ds(base, _BPW)], uidx_v)
        pltpu.sync_copy(iid_hbm.at[pl.ds(base, _BPW)], iidx_v)
        c0 = pltpu.async_copy(ug_hbm.at[uidx_v], ug_v, sem)
        c1 = pltpu.async_copy(ig_hbm.at[iidx_v], ig_v, sem)
        c2 = pltpu.async_copy(um_hbm.at[uidx_v], um_v, sem)
        c3 = pltpu.async_copy(im_hbm.at[iidx_v], im_v, sem)
        c0.wait()
        pltpu.sync_copy(ug_v, out_ug.at[pl.ds(base, _BPW)])
        c1.wait()
        pltpu.sync_copy(ig_v, out_ig.at[pl.ds(base, _BPW)])
        c2.wait()
        pltpu.sync_copy(um_v, out_um.at[pl.ds(base, _BPW)])
        c3.wait()
        pltpu.sync_copy(im_v, out_im.at[pl.ds(base, _BPW)])

    return gather_kernel(user_ids, item_ids, user_gmf, item_gmf,
                         user_mlp, item_mlp)


_BLK = 2048


def _dense_body(gu_ref, gi_ref, mu_ref, mi_ref, gn_ref, gW_ref, gb_ref,
                W1_ref, b1_ref, W2_ref, b2_ref, Wf_ref, bf_ref, out_ref):
    ge = jnp.dot(gn_ref[:], gW_ref[:],
                 preferred_element_type=jnp.float32) + gb_ref[:]
    h = (jnp.dot(mu_ref[:], W1_ref[0:D_MLP, :],
                 preferred_element_type=jnp.float32)
         + jnp.dot(mi_ref[:], W1_ref[D_MLP:2 * D_MLP, :],
                   preferred_element_type=jnp.float32)
         + jnp.dot(ge, W1_ref[2 * D_MLP:2 * D_MLP + GP, :],
                   preferred_element_type=jnp.float32)
         + b1_ref[:])
    h = jnp.maximum(h, 0.0)
    h2 = jnp.maximum(
        jnp.dot(h, W2_ref[:], preferred_element_type=jnp.float32) + b2_ref[:],
        0.0)
    gmf = gu_ref[:] * gi_ref[:]
    out_ref[:] = (jnp.dot(gmf, Wf_ref[0:D_GMF, :],
                          preferred_element_type=jnp.float32)
                  + jnp.dot(h2, Wf_ref[D_GMF:D_GMF + H2, :],
                            preferred_element_type=jnp.float32)
                  + bf_ref[:])


def _dense(gmf_u, gmf_i, mlp_u, mlp_i, genres, genre_W, genre_b,
           W1, b1, W2, b2, Wf, bf):
    grid = (B // _BLK,)
    row = lambda i: (i, 0)
    rep = lambda i: (0, 0)
    out = pl.pallas_call(
        _dense_body,
        grid=grid,
        in_specs=[
            pl.BlockSpec((_BLK, D_GMF), row),
            pl.BlockSpec((_BLK, D_GMF), row),
            pl.BlockSpec((_BLK, D_MLP), row),
            pl.BlockSpec((_BLK, D_MLP), row),
            pl.BlockSpec((_BLK, NG), row),
            pl.BlockSpec((NG, GP), rep),
            pl.BlockSpec((1, GP), rep),
            pl.BlockSpec((2 * D_MLP + GP, H1), rep),
            pl.BlockSpec((1, H1), rep),
            pl.BlockSpec((H1, H2), rep),
            pl.BlockSpec((1, H2), rep),
            pl.BlockSpec((D_GMF + H2, 1), rep),
            pl.BlockSpec((1, 1), rep),
        ],
        out_specs=pl.BlockSpec((_BLK, 1), row),
        out_shape=jax.ShapeDtypeStruct((B, 1), jnp.float32),
    )(gmf_u, gmf_i, mlp_u, mlp_i, genres,
      genre_W, genre_b.reshape(1, GP),
      W1, b1.reshape(1, H1), W2, b2.reshape(1, H2),
      Wf, bf.reshape(1, 1))
    return out[:, 0]


def kernel(user_ids, item_ids, genres, user_gmf, item_gmf, user_mlp,
           item_mlp, genre_W, genre_b, W1, b1, W2, b2, Wf, bf):
    gmf_u, gmf_i, mlp_u, mlp_i = _sc_gather(
        user_ids, item_ids, user_gmf, item_gmf, user_mlp, item_mlp)
    return _dense(gmf_u, gmf_i, mlp_u, mlp_i, genres, genre_W, genre_b,
                  W1, b1, W2, b2, Wf, bf)
